# baseline (speedup 1.0000x reference)
import functools

import jax
import jax.numpy as jnp
from jax import lax
from jax.experimental import pallas as pl
from jax.experimental.pallas import tpu as pltpu
from jax.experimental.pallas import tpu_sc as plsc

N_ROWS = 1000
D_OUT = 1001
D_PAD = 1008
B_TOTAL = 4096 * 20


def _fuse_body(emb_ref, w_ref, b_ref, t_ref):
    t_ref[...] = lax.dot_general(
        emb_ref[...], w_ref[...],
        (((1,), (1,)), ((), ())),
        preferred_element_type=jnp.float32,
    ) + b_ref[...]


def _fuse_table(emb, W, b):
    W_pad = jnp.pad(W, ((0, D_PAD - D_OUT), (0, 0)))
    b_pad = jnp.pad(b, (0, D_PAD - D_OUT)).reshape(1, D_PAD)
    return pl.pallas_call(
        _fuse_body,
        out_shape=jax.ShapeDtypeStruct((N_ROWS, D_PAD), jnp.float32),
    )(emb, W_pad, b_pad)


def _make_gather(B, D, chunk):
    info = plsc.get_sparse_core_info()
    nc, ns = info.num_cores, info.num_subcores
    nw = nc * ns
    b_per_w = B // nw
    n_chunks = b_per_w // chunk
    assert b_per_w % chunk == 0 and chunk % 8 == 0 and b_per_w % 8 == 0

    mesh = plsc.VectorSubcoreMesh(core_axis_name="c", subcore_axis_name="s")

    @functools.partial(
        pl.kernel,
        out_type=jax.ShapeDtypeStruct((B, D), jnp.float32),
        mesh=mesh,
        scratch_types=[
            pltpu.VMEM((b_per_w,), jnp.int32),
            pltpu.VMEM((chunk, D), jnp.float32),
            pltpu.SemaphoreType.DMA,
        ],
        compiler_params=pltpu.CompilerParams(use_tc_tiling_on_sc=False),
    )
    def gather_kernel(table_hbm, idx_hbm, out_hbm, idx_v, rows_v, sem):
        wid = lax.axis_index("s") * nc + lax.axis_index("c")
        base = wid * b_per_w
        pltpu.sync_copy(idx_hbm.at[pl.ds(base, b_per_w)], idx_v)

        def chunk_body(i, carry):
            cbase = i * chunk
            pltpu.async_copy(
                table_hbm.at[idx_v.at[pl.ds(cbase, chunk)]], rows_v, sem
            ).wait()
            pltpu.sync_copy(rows_v, out_hbm.at[pl.ds(base + cbase, chunk)])
            return carry

        lax.fori_loop(0, n_chunks, chunk_body, 0)

    return gather_kernel


_gather = _make_gather(B_TOTAL, D_PAD, chunk=40)


def kernel(x, emb, W, b):
    table = _fuse_table(emb, W, b)
    idx = x.reshape(-1).astype(jnp.int32)
    out = _gather(table, idx)
    return out[:, :D_OUT].reshape(x.shape[0], x.shape[1], D_OUT)


# speedup vs baseline: 1.4234x; 1.4234x over previous
import functools

import jax
import jax.numpy as jnp
from jax import lax
from jax.experimental import pallas as pl
from jax.experimental.pallas import tpu as pltpu
from jax.experimental.pallas import tpu_sc as plsc

N_ROWS = 1000
D_OUT = 1001
D_PAD = 1024
B_TOTAL = 4096 * 20


def _fuse_body(emb_ref, w_ref, b_ref, t_ref):
    t_ref[...] = lax.dot_general(
        emb_ref[...], w_ref[...],
        (((1,), (1,)), ((), ())),
        preferred_element_type=jnp.float32,
    ) + b_ref[...]


def _fuse_table(emb, W, b):
    W_pad = jnp.pad(W, ((0, D_PAD - D_OUT), (0, 0)))
    b_pad = jnp.pad(b, (0, D_PAD - D_OUT)).reshape(1, D_PAD)
    return pl.pallas_call(
        _fuse_body,
        out_shape=jax.ShapeDtypeStruct((N_ROWS, D_PAD), jnp.float32),
    )(emb, W_pad, b_pad)


def _make_gather(B, D, chunk):
    info = plsc.get_sparse_core_info()
    nc, ns = info.num_cores, info.num_subcores
    nw = nc * ns
    b_per_w = B // nw
    n_chunks = b_per_w // chunk
    assert b_per_w % chunk == 0 and chunk % 8 == 0

    mesh = plsc.VectorSubcoreMesh(core_axis_name="c", subcore_axis_name="s")

    @functools.partial(
        pl.kernel,
        out_type=jax.ShapeDtypeStruct((B, D), jnp.float32),
        mesh=mesh,
        scratch_types=[
            pltpu.VMEM((n_chunks, chunk), jnp.int32),
            pltpu.VMEM((chunk, D), jnp.float32),
            pltpu.SemaphoreType.DMA,
        ],
    )
    def gather_kernel(table_hbm, idx_hbm, out_hbm, idx_v, rows_v, sem):
        wid = lax.axis_index("s") * nc + lax.axis_index("c")
        base = wid * b_per_w
        pltpu.sync_copy(idx_hbm.at[pl.ds(wid * n_chunks, n_chunks)], idx_v)

        def chunk_body(i, carry):
            pltpu.async_copy(
                table_hbm.at[idx_v.at[i]], rows_v, sem
            ).wait()
            pltpu.sync_copy(rows_v, out_hbm.at[pl.ds(base + i * chunk, chunk)])
            return carry

        lax.fori_loop(0, n_chunks, chunk_body, 0)

    return gather_kernel


_CHUNK = 40
_gather = _make_gather(B_TOTAL, D_PAD, _CHUNK)


def kernel(x, emb, W, b):
    table = _fuse_table(emb, W, b)
    idx = x.reshape(B_TOTAL // _CHUNK, _CHUNK).astype(jnp.int32)
    out = _gather(table, idx)
    return out[:, :D_OUT].reshape(x.shape[0], x.shape[1], D_OUT)


# speedup vs baseline: 2.4309x; 1.7078x over previous
import functools

import jax
import jax.numpy as jnp
from jax import lax
from jax.experimental import pallas as pl
from jax.experimental.pallas import tpu as pltpu
from jax.experimental.pallas import tpu_sc as plsc

N_ROWS = 1000
D_OUT = 1001
D_PAD = 1024
B_TOTAL = 4096 * 20


def _fuse_body(emb_ref, w_ref, b_ref, t_ref):
    t_ref[...] = lax.dot_general(
        emb_ref[...], w_ref[...],
        (((1,), (1,)), ((), ())),
        preferred_element_type=jnp.float32,
    ) + b_ref[...]


def _fuse_table(emb, W, b):
    W_pad = jnp.pad(W, ((0, D_PAD - D_OUT), (0, 0)))
    b_pad = jnp.pad(b, (0, D_PAD - D_OUT)).reshape(1, D_PAD)
    return pl.pallas_call(
        _fuse_body,
        out_shape=jax.ShapeDtypeStruct((N_ROWS, D_PAD), jnp.float32),
    )(emb, W_pad, b_pad)


def _make_gather(B, D, chunk):
    info = plsc.get_sparse_core_info()
    nc, ns = info.num_cores, info.num_subcores
    nw = nc * ns
    b_per_w = B // nw
    n_chunks = b_per_w // chunk
    assert b_per_w % chunk == 0 and chunk % 8 == 0

    mesh = plsc.VectorSubcoreMesh(core_axis_name="c", subcore_axis_name="s")

    @functools.partial(
        pl.kernel,
        out_type=jax.ShapeDtypeStruct((B, D), jnp.float32),
        mesh=mesh,
        scratch_types=[
            pltpu.VMEM((n_chunks, chunk), jnp.int32),
            pltpu.VMEM((chunk, D), jnp.float32),
            pltpu.SemaphoreType.DMA,
        ],
    )
    def gather_kernel(table_hbm, idx_hbm, out_hbm, idx_v, rows_v, sem):
        wid = lax.axis_index("s") * nc + lax.axis_index("c")
        base = wid * b_per_w
        pltpu.sync_copy(idx_hbm.at[pl.ds(wid * n_chunks, n_chunks)], idx_v)

        def chunk_body(i, carry):
            pltpu.async_copy(
                table_hbm.at[idx_v.at[i]], rows_v, sem
            ).wait()
            pltpu.sync_copy(rows_v, out_hbm.at[pl.ds(base + i * chunk, chunk)])
            return carry

        lax.fori_loop(0, n_chunks, chunk_body, 0)

    return gather_kernel


_CHUNK = 40
_gather = _make_gather(B_TOTAL, D_PAD, _CHUNK)

_N_ITEMS = 4096
_T_STEPS = 20
_NBLK = 128


def _transpose_body(in_ref, out_ref):
    x = in_ref[...].reshape(_NBLK, _T_STEPS, D_PAD)
    for j in range(_T_STEPS):
        out_ref[j] = x[:, j, :].T[:D_OUT]


def _transpose_pack(flat):
    grid = _N_ITEMS // _NBLK
    return pl.pallas_call(
        _transpose_body,
        grid=(grid,),
        in_specs=[pl.BlockSpec((_NBLK * _T_STEPS, D_PAD), lambda g: (g, 0))],
        out_specs=pl.BlockSpec((_T_STEPS, D_OUT, _NBLK), lambda g: (0, 0, g)),
        out_shape=jax.ShapeDtypeStruct((_T_STEPS, D_OUT, _N_ITEMS), jnp.float32),
        compiler_params=pltpu.CompilerParams(
            vmem_limit_bytes=100 * 1024 * 1024
        ),
    )(flat)


def kernel(x, emb, W, b):
    table = _fuse_table(emb, W, b)
    idx = x.reshape(B_TOTAL // _CHUNK, _CHUNK).astype(jnp.int32)
    flat = _gather(table, idx)
    out_t = _transpose_pack(flat)
    return jnp.transpose(out_t, (2, 0, 1))


# speedup vs baseline: 2.5862x; 1.0639x over previous
import functools

import jax
import jax.numpy as jnp
from jax import lax
from jax.experimental import pallas as pl
from jax.experimental.pallas import tpu as pltpu
from jax.experimental.pallas import tpu_sc as plsc

N_ROWS = 1000
D_OUT = 1001
D_PAD = 1024
B_TOTAL = 4096 * 20


def _fuse_body(emb_ref, w_ref, b_ref, t_ref):
    t_ref[...] = lax.dot_general(
        emb_ref[...], w_ref[...],
        (((1,), (1,)), ((), ())),
        preferred_element_type=jnp.float32,
    ) + b_ref[...]


def _fuse_table(emb, W, b):
    W_pad = jnp.pad(W, ((0, D_PAD - D_OUT), (0, 0)))
    b_pad = jnp.pad(b, (0, D_PAD - D_OUT)).reshape(1, D_PAD)
    return pl.pallas_call(
        _fuse_body,
        out_shape=jax.ShapeDtypeStruct((N_ROWS, D_PAD), jnp.float32),
    )(emb, W_pad, b_pad)


def _make_gather(B, D, chunk):
    info = plsc.get_sparse_core_info()
    nc, ns = info.num_cores, info.num_subcores
    nw = nc * ns
    b_per_w = B // nw
    n_chunks = b_per_w // chunk
    assert b_per_w % chunk == 0 and chunk % 8 == 0

    mesh = plsc.VectorSubcoreMesh(core_axis_name="c", subcore_axis_name="s")

    assert n_chunks % 2 == 0

    @functools.partial(
        pl.kernel,
        out_type=jax.ShapeDtypeStruct((B, D), jnp.float32),
        mesh=mesh,
        scratch_types=[
            pltpu.VMEM((n_chunks, chunk), jnp.int32),
            pltpu.VMEM((chunk, D), jnp.float32),
            pltpu.VMEM((chunk, D), jnp.float32),
            pltpu.SemaphoreType.DMA,
            pltpu.SemaphoreType.DMA,
        ],
    )
    def gather_kernel(table_hbm, idx_hbm, out_hbm, idx_v, rows0, rows1, g0, g1):
        wid = lax.axis_index("s") * nc + lax.axis_index("c")
        base = wid * b_per_w
        pltpu.sync_copy(idx_hbm.at[pl.ds(wid * n_chunks, n_chunks)], idx_v)

        pltpu.async_copy(table_hbm.at[idx_v.at[0]], rows0, g0)
        pltpu.async_copy(table_hbm.at[idx_v.at[1]], rows1, g1)

        def pair_body(k, carry):
            i = 2 * k

            def do_half(i, rows, sem):
                pltpu.make_async_copy(
                    table_hbm.at[idx_v.at[i]], rows, sem
                ).wait()
                pltpu.sync_copy(
                    rows, out_hbm.at[pl.ds(base + i * chunk, chunk)]
                )

                @pl.when(i + 2 < n_chunks)
                def _():
                    pltpu.async_copy(
                        table_hbm.at[idx_v.at[i + 2]], rows, sem
                    )

            do_half(i, rows0, g0)
            do_half(i + 1, rows1, g1)
            return carry

        lax.fori_loop(0, n_chunks // 2, pair_body, 0)

    return gather_kernel


_CHUNK = 40
_gather = _make_gather(B_TOTAL, D_PAD, _CHUNK)

_N_ITEMS = 4096
_T_STEPS = 20
_NBLK = 128


def _transpose_body(in_ref, out_ref):
    x = in_ref[...].reshape(_NBLK, _T_STEPS, D_PAD)
    for j in range(_T_STEPS):
        out_ref[j] = x[:, j, :].T[:D_OUT]


def _transpose_pack(flat):
    grid = _N_ITEMS // _NBLK
    return pl.pallas_call(
        _transpose_body,
        grid=(grid,),
        in_specs=[pl.BlockSpec((_NBLK * _T_STEPS, D_PAD), lambda g: (g, 0))],
        out_specs=pl.BlockSpec((_T_STEPS, D_OUT, _NBLK), lambda g: (0, 0, g)),
        out_shape=jax.ShapeDtypeStruct((_T_STEPS, D_OUT, _N_ITEMS), jnp.float32),
        compiler_params=pltpu.CompilerParams(
            vmem_limit_bytes=100 * 1024 * 1024
        ),
    )(flat)


def kernel(x, emb, W, b):
    table = _fuse_table(emb, W, b)
    idx = x.reshape(B_TOTAL // _CHUNK, _CHUNK).astype(jnp.int32)
    flat = _gather(table, idx)
    out_t = _transpose_pack(flat)
    return jnp.transpose(out_t, (2, 0, 1))
